# baseline (speedup 1.0000x reference)
import functools

import jax
import jax.numpy as jnp
from jax import lax
from jax.experimental import pallas as pl
from jax.experimental.pallas import tpu as pltpu
from jax.experimental.pallas import tpu_sc as plsc

_LANES = 16
_NBUF = 2


def _gather_splits(s: int):
    splits, off = [], 0
    while off < s:
        n = min(128, s - off)
        splits.append((off, n))
        off += n
    assert all(o % 8 == 0 for o, _ in splits)
    return splits


@functools.lru_cache(maxsize=None)
def _build_sc_kernel(b_total: int, s_len: int, n_ctx: int, d: int,
                     nc: int, ns: int):
    nw = nc * ns
    rows_per_w = b_total // nw
    d_vecs = d // _LANES
    splits = _gather_splits(s_len)

    mesh = plsc.VectorSubcoreMesh(core_axis_name="c", subcore_axis_name="s")

    @functools.partial(
        pl.kernel,
        out_type=jax.ShapeDtypeStruct((b_total, s_len, d), jnp.float32),
        mesh=mesh,
        scratch_types=[
            pltpu.VMEM((n_ctx, d), jnp.float32),
            pltpu.VMEM((2, d), jnp.float32),
            pltpu.VMEM((2 * n_ctx, d), jnp.float32),
            pltpu.VMEM_SHARED((2 * n_ctx, d), jnp.float32),
            [pltpu.VMEM((s_len,), jnp.int32)] * _NBUF,
            [pltpu.VMEM((s_len,), jnp.int32)] * _NBUF,
            [pltpu.VMEM((s_len,), jnp.int32)] * _NBUF,
            [pltpu.VMEM((s_len, d), jnp.float32)] * _NBUF,
            [pltpu.SemaphoreType.DMA] * _NBUF,
            [pltpu.SemaphoreType.DMA] * _NBUF,
            [pltpu.SemaphoreType.DMA] * _NBUF,
        ],
        compiler_params=pltpu.CompilerParams(use_tc_tiling_on_sc=False),
    )
    def sc_kernel(pos_hbm, seq_hbm, ptab_hbm, stab_hbm, out_hbm,
                  ptab_v, stab_v, comb_v, comb_sh, posv, seqv, idxv, outv,
                  sem_in, sem_g, sem_w):
        c = lax.axis_index("c")
        s = lax.axis_index("s")
        wid = c * ns + s
        w_base = wid * rows_per_w

        @pl.when(s == 0)
        def _build():
            pltpu.sync_copy(ptab_hbm, ptab_v)
            pltpu.sync_copy(stab_hbm, stab_v)

            def row(p, carry):
                for dc in range(d_vecs):
                    sl = pl.ds(dc * _LANES, _LANES)
                    v = ptab_v[p, sl]
                    comb_v[p, sl] = v + stab_v[0, sl]
                    comb_v[n_ctx + p, sl] = v + stab_v[1, sl]
                return carry

            lax.fori_loop(0, n_ctx, row, 0)
            pltpu.sync_copy(comb_v, comb_sh)

        plsc.subcore_barrier()

        def load_idx(m, b):
            bi = w_base + m
            pltpu.async_copy(pos_hbm.at[bi], posv[b], sem_in[b])
            pltpu.async_copy(seq_hbm.at[bi], seqv[b], sem_in[b])

        def drain_idx(m, b):
            bi = w_base + m
            pltpu.make_async_copy(pos_hbm.at[bi], posv[b], sem_in[b]).wait()
            pltpu.make_async_copy(seq_hbm.at[bi], seqv[b], sem_in[b]).wait()

        for b in range(_NBUF):
            load_idx(b, b)

        def super_body(m2, carry):
            for b in range(_NBUF):
                m = m2 * _NBUF + b
                drain_idx(m, b)

                def fuse(i, carry2):
                    sl = pl.ds(i * _LANES, _LANES)
                    idxv[b][sl] = seqv[b][sl] * n_ctx + posv[b][sl]
                    return carry2

                lax.fori_loop(0, s_len // _LANES, fuse, 0)

                @pl.when(m + _NBUF < rows_per_w)
                def _prefetch():
                    load_idx(m + _NBUF, b)

                @pl.when(m2 > 0)
                def _drain_wb():
                    pltpu.make_async_copy(
                        outv[b], out_hbm.at[w_base + m - _NBUF],
                        sem_w[b]).wait()

                copies = []
                for off, n in splits:
                    gsl = pl.ds(off, n)
                    copies.append(
                        pltpu.async_copy(comb_sh.at[idxv[b].at[gsl]],
                                         outv[b].at[gsl], sem_g[b]))
                for cp in copies:
                    cp.wait()

                pltpu.async_copy(outv[b], out_hbm.at[w_base + m], sem_w[b])
            return carry

        lax.fori_loop(0, rows_per_w // _NBUF, super_body, 0)

        for b in range(_NBUF):
            pltpu.make_async_copy(
                outv[b], out_hbm.at[w_base + rows_per_w - _NBUF + b],
                sem_w[b]).wait()

    return sc_kernel


def kernel(positions, sequence_ids, pos_table, seq_table):
    b, s = positions.shape
    n_ctx, d = pos_table.shape

    info = plsc.get_sparse_core_info()
    nc, ns = info.num_cores, info.num_subcores
    nw = nc * ns

    assert b % (nw * _NBUF) == 0 and d % _LANES == 0 and s % _LANES == 0

    sc = _build_sc_kernel(b, s, n_ctx, d, nc, ns)
    return sc(positions.astype(jnp.int32), sequence_ids.astype(jnp.int32),
              pos_table.astype(jnp.float32), seq_table.astype(jnp.float32))
